# baseline (speedup 1.0000x reference)
import functools

import jax
import jax.numpy as jnp
from jax import lax
from jax.experimental import pallas as pl
from jax.experimental.pallas import tpu as pltpu
from jax.experimental.pallas import tpu_sc as plsc

GROUPS = 4
V = 1024
B = 1024
D = 256
EG = 8
SR = 16
NSL = D // SR
NC = 2
NS = 16
L = 16
NW = NC * NS
BPW = B // NW
NDMA = (BPW // EG) * NSL

_mesh = plsc.VectorSubcoreMesh(core_axis_name="c", subcore_axis_name="s")


@functools.partial(
    pl.kernel,
    mesh=_mesh,
    out_type=(
        jax.ShapeDtypeStruct((B, D, D), jnp.float32),
        jax.ShapeDtypeStruct((B, D), jnp.float32),
    ),
    scratch_types=[
        pltpu.VMEM((L,), jnp.int32),
        pltpu.VMEM((L,), jnp.int32),
        pltpu.VMEM((1, B), jnp.int32),
        pltpu.VMEM((BPW, D), jnp.float32),
        pltpu.VMEM((EG, SR, D), jnp.float32),
        pltpu.VMEM((EG, SR, D), jnp.float32),
        pltpu.VMEM((EG, SR, D), jnp.float32),
        pltpu.SemaphoreType.DMA,
        pltpu.SemaphoreType.DMA,
        pltpu.SemaphoreType.DMA,
        pltpu.SemaphoreType.DMA,
        pltpu.SemaphoreType.DMA,
        pltpu.SemaphoreType.DMA,
        pltpu.SemaphoreType.DMA,
    ],
)
def _sc_lookup(zeros16, ld, selT, wtab, btab, outw, outb,
               z_v, g_v, widx_v, bbuf, wbuf0, wbuf1, wbuf2,
               usem, gs0, gs1, gs2, ps0, ps1, ps2):
    wid = lax.axis_index("s") * NC + lax.axis_index("c")
    base = pl.multiple_of(wid * BPW, BPW)

    pltpu.sync_copy(zeros16, z_v)
    pltpu.async_copy(ld.at[z_v], g_v, usem).wait()
    pltpu.async_copy(selT.at[g_v.at[pl.ds(0, 1)]], widx_v, usem).wait()

    b_gather = pltpu.make_async_copy(
        btab.at[widx_v.at[0, pl.ds(base, BPW)]], bbuf, usem)
    b_gather.start()

    bufs = (wbuf0, wbuf1, wbuf2)
    gsems = (gs0, gs1, gs2)
    psems = (ps0, ps1, ps2)

    def g_desc(d, b):
        o = pl.multiple_of((d // NSL) * EG, EG)
        c = pl.multiple_of((d % NSL) * SR, SR)
        return pltpu.make_async_copy(
            wtab.at[widx_v.at[0, pl.ds(base + o, EG)], pl.ds(c, SR)],
            bufs[b], gsems[b])

    def p_desc(d, b):
        o = pl.multiple_of((d // NSL) * EG, EG)
        c = pl.multiple_of((d % NSL) * SR, SR)
        return pltpu.make_async_copy(
            bufs[b], outw.at[pl.ds(base + o, EG), pl.ds(c, SR)], psems[b])

    g_desc(0, 0).start()
    g_desc(1, 1).start()

    def step(i, carry):
        for k in range(3):
            d = i * 3 + k
            g_desc(d, k).wait()
            p_desc(d, k).start()
            dn = d + 2
            bn = (k + 2) % 3

            @pl.when(dn < NDMA)
            def _():
                @pl.when(d >= 1)
                def _():
                    p_desc(d - 1, bn).wait()

                g_desc(dn, bn).start()
        return carry

    lax.fori_loop(0, (NDMA - 1) // 3, step, 0)
    g_desc(NDMA - 1, 0).wait()
    p_desc(NDMA - 1, 0).start()
    b_gather.wait()
    pltpu.sync_copy(bbuf, outb.at[pl.ds(base, BPW)])
    p_desc(NDMA - 3, 1).wait()
    p_desc(NDMA - 2, 2).wait()
    p_desc(NDMA - 1, 0).wait()


def kernel(layer_index, selector, W0, b0, layers_distribution):
    ld = lax.dynamic_slice_in_dim(layers_distribution, layer_index, 1)
    selT = selector.T
    zeros16 = jnp.zeros((L,), jnp.int32)
    outw, outb = _sc_lookup(zeros16, ld, selT, W0, b0)
    return (outw, outb)
